# baseline (speedup 1.0000x reference)
import functools

import jax
import jax.numpy as jnp
from jax import lax
from jax.experimental import pallas as pl
from jax.experimental.pallas import tpu as pltpu
from jax.experimental.pallas import tpu_sc as plsc

VOCAB = 1000000
EMBED = 64
B = 16384
L = 50
HID = 50
NCLS = 2

NC = 2
NS = 16
NW = NC * NS
ROWS_W = B // NW
NV = EMBED // 16
NBUF = 4


WIN = 512
NWIN = VOCAB // WIN
WPW = NWIN // NW + 1


NCB = 2


def _cv_body(tt_hbm, tail_hbm, out_hbm, wins, outs, isems, osems):
    wid = lax.axis_index("s") * NC + lax.axis_index("c")
    iota = lax.iota(jnp.int32, 16)

    def win_start(wi):
        return pl.multiple_of(wi * WIN, WIN)

    def fetch(wi, j):
        @pl.when(wi < NWIN)
        def _():
            for t in range(8):
                pltpu.async_copy(
                    tt_hbm.at[pl.ds(8 * t, 8), pl.ds(win_start(wi), WIN)],
                    wins[j].at[pl.ds(8 * t, 8), pl.ds(0, WIN)], isems[j])

    def wait_fetch(wi, j):
        for t in range(8):
            pltpu.make_async_copy(
                tt_hbm.at[pl.ds(8 * t, 8), pl.ds(win_start(wi), WIN)],
                wins[j].at[pl.ds(8 * t, 8), pl.ds(0, WIN)], isems[j]).wait()

    def transpose(j, o):
        rowv = [r * 16 + iota for r in range(4)]

        def qbody(qq, carry):
            colA, colB = carry
            for u in range(4):
                vals = [plsc.load_gather(wins[j], [rowv[k % 4], colA if k < 4 else colB])
                        for k in range(8)]
                for k in range(8):
                    outs[o][qq * 4 + u, pl.ds(k * 16, 16)] = vals[k]
                colA = colA + 2
                colB = colB + 2
            return (colA, colB)

        lax.fori_loop(0, 32, qbody,
                      (jnp.full((16,), 256 * o, jnp.int32),
                       jnp.full((16,), 256 * o + 1, jnp.int32)))

    for j in range(NCB):
        fetch(wid + j * NW, j)

    def body(i, _):
        for j in range(NCB):
            wi = (NCB * i + j) * NW + wid

            @pl.when(wi < NWIN)
            def _():
                wait_fetch(wi, j)
                for o in range(2):
                    @pl.when(wi >= NW)
                    def _():
                        pltpu.make_async_copy(
                            outs[o], out_hbm.at[pl.ds((wi - NW) * 256 + o * 128, 128)],
                            osems[o]).wait()

                    transpose(j, o)
                    pltpu.async_copy(
                        outs[o], out_hbm.at[pl.ds(wi * 256 + o * 128, 128)], osems[o])
                fetch(wi + NCB * NW, j)

        return 0

    lax.fori_loop(0, (WPW + NCB - 1) // NCB, body, 0)

    for o in range(2):
        pltpu.make_async_copy(out_hbm.at[pl.ds(0, 128)], outs[o], osems[o]).wait()

    @pl.when(wid == 0)
    def _():
        pltpu.sync_copy(tail_hbm, outs[0].at[pl.ds(0, 32)])
        pltpu.sync_copy(outs[0].at[pl.ds(0, 32)], out_hbm.at[pl.ds(NWIN * 256, 32)])


def _convert(tt, tail):
    mesh = plsc.VectorSubcoreMesh(core_axis_name="c", subcore_axis_name="s")
    return pl.kernel(
        _cv_body,
        out_type=jax.ShapeDtypeStruct((VOCAB // 2, 2 * EMBED), jnp.float32),
        mesh=mesh,
        scratch_types=[
            [pltpu.VMEM((EMBED, WIN + 1), jnp.float32) for _ in range(NCB)],
            [pltpu.VMEM((WIN // 4, 2 * EMBED), jnp.float32) for _ in range(NCB)],
            [pltpu.SemaphoreType.DMA for _ in range(NCB)],
            [pltpu.SemaphoreType.DMA for _ in range(NCB)],
        ],
        compiler_params=pltpu.CompilerParams(
            use_tc_tiling_on_sc=True, needs_layout_passes=False,
            disable_bounds_checks=True),
    )(tt, tail)


def _sc_body(text_hbm, table_hbm, out_hbm, idx_v, bufs, out_v, sems):
    wid = lax.axis_index("s") * NC + lax.axis_index("c")
    pltpu.sync_copy(text_hbm.at[pl.ds(wid * ROWS_W, ROWS_W)], idx_v)

    def accumulate(buf, b):
        accs = [buf[0, pl.ds(k * 16, 16)] for k in range(NV)]
        for l in range(1, L):
            for k in range(NV):
                accs[k] = accs[k] + buf[l, pl.ds(k * 16, 16)]
        for k in range(NV):
            out_v[b, pl.ds(k * 16, 16)] = accs[k] * (1.0 / L)

    for j in range(NBUF):
        pltpu.async_copy(table_hbm.at[idx_v.at[j]], bufs[j], sems[j])

    def body(gp, _):
        for j in range(NBUF):
            b = gp * NBUF + j
            pltpu.make_async_copy(table_hbm.at[idx_v.at[b]], bufs[j], sems[j]).wait()
            accumulate(bufs[j], b)
            nxt = b + NBUF

            @pl.when(nxt < ROWS_W)
            def _():
                pltpu.async_copy(table_hbm.at[idx_v.at[nxt]], bufs[j], sems[j])

        return 0

    lax.fori_loop(0, ROWS_W // NBUF, body, 0)

    pltpu.sync_copy(out_v, out_hbm.at[pl.ds(wid * ROWS_W, ROWS_W)])


def _sc_pool(text, table):
    mesh = plsc.VectorSubcoreMesh(core_axis_name="c", subcore_axis_name="s")
    return pl.kernel(
        _sc_body,
        out_type=jax.ShapeDtypeStruct((B, EMBED), jnp.float32),
        mesh=mesh,
        scratch_types=[
            pltpu.VMEM((ROWS_W, L), jnp.int32),
            [pltpu.VMEM((L, EMBED), jnp.float32) for _ in range(NBUF)],
            pltpu.VMEM((ROWS_W, EMBED), jnp.float32),
            [pltpu.SemaphoreType.DMA for _ in range(NBUF)],
        ],
        compiler_params=pltpu.CompilerParams(
            use_tc_tiling_on_sc=False, needs_layout_passes=False,
            disable_bounds_checks=True),
    )(text, table)


def _mlp_body(feat_ref, w1t_ref, b1_ref, w2t_ref, b2_ref, out_ref):
    h = jnp.dot(feat_ref[...], w1t_ref[...], preferred_element_type=jnp.float32)
    h = jnp.maximum(h + b1_ref[...], 0.0)
    out_ref[...] = jnp.dot(h, w2t_ref[...], preferred_element_type=jnp.float32) + b2_ref[...]


def _mlp(feat, w1t, b1r, w2t, b2r):
    return pl.pallas_call(
        _mlp_body,
        out_shape=jax.ShapeDtypeStruct((B, NCLS), jnp.float32),
    )(feat, w1t, b1r, w2t, b2r)


def kernel(text, text_len, table, W1, b1, W2, b2):
    tt = table.T
    tailp = table[VOCAB - 64:].reshape(32, 2 * EMBED)
    tablep = _convert(tt, tailp)
    pooled = _sc_pool(text, tablep.reshape(VOCAB, EMBED))

    len_col = text_len.astype(jnp.float32).reshape(B, 1)
    feat = jnp.concatenate([pooled, len_col], axis=1)
    out = _mlp(feat, W1.T, b1.reshape(1, HID), W2.T, b2.reshape(1, NCLS))
    return out


# speedup vs baseline: 1.6099x; 1.6099x over previous
import functools

import jax
import jax.numpy as jnp
from jax import lax
from jax.experimental import pallas as pl
from jax.experimental.pallas import tpu as pltpu
from jax.experimental.pallas import tpu_sc as plsc

VOCAB = 1000000
EMBED = 64
B = 16384
L = 50
HID = 50
NCLS = 2

NC = 2
NS = 16
NW = NC * NS
ROWS_W = B // NW
NV = EMBED // 16
NBUF = 4


def _sc_body(text_hbm, table_hbm, out_hbm, idx_v, bufs, out_v, sems):
    wid = lax.axis_index("s") * NC + lax.axis_index("c")
    pltpu.sync_copy(text_hbm.at[pl.ds(wid * ROWS_W, ROWS_W)], idx_v)

    def accumulate(buf, b):
        accs = [buf[0, pl.ds(k * 16, 16)] for k in range(NV)]
        for l in range(1, L):
            for k in range(NV):
                accs[k] = accs[k] + buf[l, pl.ds(k * 16, 16)]
        for k in range(NV):
            out_v[b, pl.ds(k * 16, 16)] = accs[k] * (1.0 / L)

    for j in range(NBUF):
        pltpu.async_copy(table_hbm.at[idx_v.at[j]], bufs[j], sems[j])

    def body(gp, _):
        for j in range(NBUF):
            b = gp * NBUF + j
            pltpu.make_async_copy(table_hbm.at[idx_v.at[b]], bufs[j], sems[j]).wait()
            accumulate(bufs[j], b)
            nxt = b + NBUF

            @pl.when(nxt < ROWS_W)
            def _():
                pltpu.async_copy(table_hbm.at[idx_v.at[nxt]], bufs[j], sems[j])

        return 0

    lax.fori_loop(0, ROWS_W // NBUF, body, 0)

    pltpu.sync_copy(out_v, out_hbm.at[pl.ds(wid * ROWS_W, ROWS_W)])


def _sc_pool(text, table):
    mesh = plsc.VectorSubcoreMesh(core_axis_name="c", subcore_axis_name="s")
    return pl.kernel(
        _sc_body,
        out_type=jax.ShapeDtypeStruct((B, EMBED), jnp.float32),
        mesh=mesh,
        scratch_types=[
            pltpu.VMEM((ROWS_W, L), jnp.int32),
            [pltpu.VMEM((L, EMBED), jnp.float32) for _ in range(NBUF)],
            pltpu.VMEM((ROWS_W, EMBED), jnp.float32),
            [pltpu.SemaphoreType.DMA for _ in range(NBUF)],
        ],
        compiler_params=pltpu.CompilerParams(use_tc_tiling_on_sc=False),
    )(text, table)


def _mlp_body(feat_ref, w1t_ref, b1_ref, w2t_ref, b2_ref, out_ref):
    h = jnp.dot(feat_ref[...], w1t_ref[...], preferred_element_type=jnp.float32)
    h = jnp.maximum(h + b1_ref[...], 0.0)
    out_ref[...] = jnp.dot(h, w2t_ref[...], preferred_element_type=jnp.float32) + b2_ref[...]


def _mlp(feat, w1t, b1r, w2t, b2r):
    return pl.pallas_call(
        _mlp_body,
        out_shape=jax.ShapeDtypeStruct((B, NCLS), jnp.float32),
    )(feat, w1t, b1r, w2t, b2r)


def kernel(text, text_len, table, W1, b1, W2, b2):
    pooled = _sc_pool(text, table)

    len_col = text_len.astype(jnp.float32).reshape(B, 1)
    feat = jnp.concatenate([pooled, len_col], axis=1)
    out = _mlp(feat, W1.T, b1.reshape(1, HID), W2.T, b2.reshape(1, NCLS))
    return out


# speedup vs baseline: 1.6232x; 1.0083x over previous
import functools

import jax
import jax.numpy as jnp
from jax import lax
from jax.experimental import pallas as pl
from jax.experimental.pallas import tpu as pltpu
from jax.experimental.pallas import tpu_sc as plsc

VOCAB = 1000000
EMBED = 64
B = 16384
L = 50
HID = 50
NCLS = 2

NC = 2
NS = 16
NW = NC * NS
ROWS_W = B // NW
NV = EMBED // 16
NBUF = 4


def _sc_body(text_hbm, table_hbm, out_hbm, idx_v, bufs, out_v, sems):
    wid = lax.axis_index("s") * NC + lax.axis_index("c")
    pltpu.sync_copy(text_hbm.at[pl.ds(wid * ROWS_W, ROWS_W)], idx_v)

    def accumulate(buf, b):
        accs = [buf[0, pl.ds(k * 16, 16)] for k in range(NV)]
        for l in range(1, L):
            for k in range(NV):
                accs[k] = accs[k] + buf[l, pl.ds(k * 16, 16)]
        for k in range(NV):
            out_v[b, pl.ds(k * 16, 16)] = accs[k] * (1.0 / L)

    for j in range(NBUF):
        pltpu.async_copy(table_hbm.at[idx_v.at[j]], bufs[j], sems[j])

    def body(gp, _):
        for j in range(NBUF):
            b = gp * NBUF + j
            pltpu.make_async_copy(table_hbm.at[idx_v.at[b]], bufs[j], sems[j]).wait()
            accumulate(bufs[j], b)
            nxt = b + NBUF

            @pl.when(nxt < ROWS_W)
            def _():
                pltpu.async_copy(table_hbm.at[idx_v.at[nxt]], bufs[j], sems[j])

        return 0

    lax.fori_loop(0, ROWS_W // NBUF, body, 0)

    pltpu.sync_copy(out_v, out_hbm.at[pl.ds(wid * ROWS_W, ROWS_W)])


def _sc_pool(text, table):
    mesh = plsc.VectorSubcoreMesh(core_axis_name="c", subcore_axis_name="s")
    return pl.kernel(
        _sc_body,
        out_type=jax.ShapeDtypeStruct((B, EMBED), jnp.float32),
        mesh=mesh,
        scratch_types=[
            pltpu.VMEM((ROWS_W, L), jnp.int32),
            [pltpu.VMEM((L, EMBED), jnp.float32) for _ in range(NBUF)],
            pltpu.VMEM((ROWS_W, EMBED), jnp.float32),
            [pltpu.SemaphoreType.DMA for _ in range(NBUF)],
        ],
        compiler_params=pltpu.CompilerParams(use_tc_tiling_on_sc=False),
    )(text, table)


def _mlp_body(pooled_ref, len_ref, w1t_ref, b1_ref, w2t_ref, b2_ref, out_ref):
    feat = jnp.concatenate([pooled_ref[...], len_ref[...]], axis=1)
    h = jnp.dot(feat, w1t_ref[...], preferred_element_type=jnp.float32)
    h = jnp.maximum(h + b1_ref[...], 0.0)
    out_ref[...] = jnp.dot(h, w2t_ref[...], preferred_element_type=jnp.float32) + b2_ref[...]


def _mlp(pooled, len_col, w1t, b1r, w2t, b2r):
    return pl.pallas_call(
        _mlp_body,
        out_shape=jax.ShapeDtypeStruct((B, NCLS), jnp.float32),
    )(pooled, len_col, w1t, b1r, w2t, b2r)


def kernel(text, text_len, table, W1, b1, W2, b2):
    pooled = _sc_pool(text, table)

    len_col = text_len.astype(jnp.float32).reshape(B, 1)
    out = _mlp(pooled, len_col, W1.T, b1.reshape(1, HID), W2.T, b2.reshape(1, NCLS))
    return out
